# Initial kernel scaffold; baseline (speedup 1.0000x reference)
import jax
import jax.numpy as jnp
from jax.experimental import pallas as pl


def kernel(x, edge_index, edge_attr, batch, node_W, node_b, eW1, eb1, eW2, eb2, conv_W, conv_b, bn_g, bn_b, rW1, rb1, rW2, rb2):
    raise NotImplementedError("write your pallas kernel here")


# speedup vs baseline: 7.8349x; 7.8349x over previous
import functools

import jax
import jax.numpy as jnp
from jax import lax
from jax.experimental import pallas as pl
from jax.experimental.pallas import tpu as pltpu
from jax.experimental.pallas import tpu_sc as plsc

N = 10000
E = 320000
H = 128
G = 64
NPAD = 10240
NW = 32
CHUNK = 128
CHUNKS_PER_W = 79
EPW = CHUNKS_PER_W * CHUNK
EPAD = NW * EPW
ROWS_PER_TILE = NPAD // 16


def _emlp_body(ea_ref, w1_ref, b1_ref, w2_ref, b2_ref, out_ref):
    a = jnp.dot(ea_ref[...], w1_ref[...], preferred_element_type=jnp.float32)
    a = a + b1_ref[...]
    a = jnp.where(a >= 0, a, 0.01 * a)
    z = jnp.sum(a * w2_ref[...], axis=1) + b2_ref[0, 0]
    out_ref[0, 0, :] = jax.nn.sigmoid(z)


def _edge_mlp(edge_attr, eW1, eb1, eW2, eb2):
    blk = 2560
    grid = E // blk
    return pl.pallas_call(
        _emlp_body,
        grid=(grid,),
        in_specs=[
            pl.BlockSpec((blk, 16), lambda i: (i, 0)),
            pl.BlockSpec((16, H), lambda i: (0, 0)),
            pl.BlockSpec((1, H), lambda i: (0, 0)),
            pl.BlockSpec((1, H), lambda i: (0, 0)),
            pl.BlockSpec((1, 1), lambda i: (0, 0)),
        ],
        out_specs=pl.BlockSpec((1, 1, blk), lambda i: (i, 0, 0)),
        out_shape=jax.ShapeDtypeStruct((grid, 1, blk), jnp.float32),
    )(edge_attr, eW1, eb1.reshape(1, H), eW2.reshape(1, H),
      eb2.reshape(1, 1)).reshape(E)


def _embed_body(x_ref, nw_ref, nb_ref, w0_ref, d0_ref, d1_ref,
                dinv_ref, tp_ref):
    deg = d0_ref[...][:N] + d1_ref[...][:N] + 1.0
    dinv = lax.rsqrt(deg)
    dinv_ref[...] = dinv
    h = jnp.dot(x_ref[...], nw_ref[...], preferred_element_type=jnp.float32)
    h = h + nb_ref[...]
    t = jnp.dot(h, w0_ref[...], preferred_element_type=jnp.float32)
    tp_ref[...] = t * dinv


def _embed(x, node_W, node_b, W0, d0, d1):
    return pl.pallas_call(
        _embed_body,
        out_shape=(
            jax.ShapeDtypeStruct((N, 1), jnp.float32),
            jax.ShapeDtypeStruct((N, H), jnp.float32),
        ),
    )(x, node_W, node_b.reshape(1, H), W0, d0, d1)


def _layer_body(s_ref, tp_ref, dinv_ref, cb_ref, g_ref, b_ref, wn_ref,
                out_ref):
    dinv = dinv_ref[...]
    agg = dinv * (s_ref[0, :N] + s_ref[1, :N] + tp_ref[...]) + cb_ref[...]
    mean = jnp.mean(agg, axis=0, keepdims=True)
    cen = agg - mean
    var = jnp.mean(cen * cen, axis=0, keepdims=True)
    hn = cen * lax.rsqrt(var + 1e-5) * g_ref[...] + b_ref[...]
    hn = jnp.where(hn >= 0, hn, 0.01 * hn)
    out_ref[...] = jnp.dot(hn, wn_ref[...],
                           preferred_element_type=jnp.float32) * dinv


def _layer(S, tp, dinv, cb, g, b, Wn):
    return pl.pallas_call(
        _layer_body,
        out_shape=jax.ShapeDtypeStruct((N, H), jnp.float32),
    )(S, tp, dinv, cb.reshape(1, H), g.reshape(1, H), b.reshape(1, H), Wn)


def _final_body(s_ref, tp_ref, dinv_ref, cb_ref, g_ref, b_ref, batch_ref,
                rw1_ref, rb1_ref, rw2_ref, rb2_ref, out_ref):
    dinv = dinv_ref[...]
    agg = dinv * (s_ref[0, :N] + s_ref[1, :N] + tp_ref[...]) + cb_ref[...]
    mean = jnp.mean(agg, axis=0, keepdims=True)
    cen = agg - mean
    var = jnp.mean(cen * cen, axis=0, keepdims=True)
    hn = cen * lax.rsqrt(var + 1e-5) * g_ref[...] + b_ref[...]
    hn = jnp.where(hn >= 0, hn, 0.01 * hn)
    oh = (lax.broadcasted_iota(jnp.int32, (G, N), 0)
          == batch_ref[...]).astype(jnp.float32)
    pooled = jnp.dot(oh, hn, preferred_element_type=jnp.float32)
    r1 = jnp.dot(pooled, rw1_ref[...], preferred_element_type=jnp.float32)
    r1 = r1 + rb1_ref[...]
    r1 = jnp.where(r1 >= 0, r1, 0.01 * r1)
    out_ref[...] = (jnp.sum(r1 * rw2_ref[...], axis=1, keepdims=True)
                    + rb2_ref[...])


def _final(S, tp, dinv, cb, g, b, batch2, rW1, rb1, rW2, rb2):
    return pl.pallas_call(
        _final_body,
        out_shape=jax.ShapeDtypeStruct((G, 1), jnp.float32),
    )(S, tp, dinv, cb.reshape(1, H), g.reshape(1, H), b.reshape(1, H),
      batch2, rW1, rb1.reshape(1, H // 2), rW2.reshape(1, H // 2),
      rb2.reshape(1, 1))


@functools.cache
def _get_sc_deg():
    return functools.partial(
        pl.kernel,
        mesh=plsc.VectorSubcoreMesh(core_axis_name="c", subcore_axis_name="s"),
        out_type=jax.ShapeDtypeStruct((2, NPAD), jnp.float32),
        scratch_types=[
            pltpu.VMEM((CHUNKS_PER_W, CHUNK), jnp.int32),
            pltpu.VMEM((CHUNKS_PER_W, CHUNK), jnp.float32),
            pltpu.VMEM_SHARED((NPAD,), jnp.float32),
        ],
    )(_sc_deg_body)


def _sc_deg_body(col_hbm, ew_hbm, zdeg_hbm, out_hbm, col_v, ew_v, deg_sh):
    c = lax.axis_index("c")
    s = lax.axis_index("s")
    w = c * 16 + s

    @pl.when(s == 0)
    def _():
        pltpu.sync_copy(zdeg_hbm, deg_sh)

    plsc.subcore_barrier()
    pltpu.sync_copy(col_hbm.at[w], col_v)
    pltpu.sync_copy(ew_hbm.at[w], ew_v)

    def chunk(j, carry):
        pltpu.sync_copy(ew_v.at[j], deg_sh.at[col_v.at[j]], add=True)
        return carry

    lax.fori_loop(0, CHUNKS_PER_W, chunk, 0)
    plsc.subcore_barrier()

    @pl.when(s == 0)
    def _():
        pltpu.sync_copy(deg_sh, out_hbm.at[c])


@functools.cache
def _get_sc_msg():
    return functools.partial(
        pl.kernel,
        mesh=plsc.VectorSubcoreMesh(core_axis_name="c", subcore_axis_name="s"),
        out_type=jax.ShapeDtypeStruct((2, NPAD, H), jnp.float32),
        scratch_types=[
            pltpu.VMEM((CHUNKS_PER_W, CHUNK), jnp.int32),
            pltpu.VMEM((CHUNKS_PER_W, CHUNK), jnp.int32),
            pltpu.VMEM((CHUNKS_PER_W, CHUNK), jnp.float32),
            pltpu.VMEM((CHUNK, H), jnp.float32),
            pltpu.VMEM_SHARED((NPAD, H), jnp.float32),
        ],
    )(_sc_msg_body)


def _sc_msg_body(tp_hbm, row_hbm, col_hbm, ew_hbm, znode_hbm, out_hbm,
                 row_v, col_v, ew_v, rows_v, acc_sh):
    c = lax.axis_index("c")
    s = lax.axis_index("s")
    w = c * 16 + s

    pltpu.sync_copy(znode_hbm.at[pl.ds(s * ROWS_PER_TILE, ROWS_PER_TILE)],
                    acc_sh.at[pl.ds(s * ROWS_PER_TILE, ROWS_PER_TILE)])
    plsc.subcore_barrier()

    pltpu.sync_copy(row_hbm.at[w], row_v)
    pltpu.sync_copy(col_hbm.at[w], col_v)
    pltpu.sync_copy(ew_hbm.at[w], ew_v)

    def chunk(j, carry):
        pltpu.sync_copy(tp_hbm.at[row_v.at[j]], rows_v)

        def group(jg, inner):
            sv16 = ew_v[j, pl.ds(jg * 16, 16)]

            for k in range(16):
                sv = sv16.at[jnp.full((16,), k, jnp.int32)].get(
                    mode="promise_in_bounds")
                e = jg * 16 + k
                for f in range(H // 16):
                    rows_v[e, pl.ds(f * 16, 16)] = (
                        rows_v[e, pl.ds(f * 16, 16)] * sv)
            return inner

        lax.fori_loop(0, CHUNK // 16, group, 0)
        pltpu.sync_copy(rows_v, acc_sh.at[col_v.at[j]], add=True)
        return carry

    lax.fori_loop(0, CHUNKS_PER_W, chunk, 0)
    plsc.subcore_barrier()

    pltpu.sync_copy(acc_sh.at[pl.ds(s * ROWS_PER_TILE, ROWS_PER_TILE)],
                    out_hbm.at[c, pl.ds(s * ROWS_PER_TILE, ROWS_PER_TILE)])


def kernel(x, edge_index, edge_attr, batch, node_W, node_b, eW1, eb1, eW2,
           eb2, conv_W, conv_b, bn_g, bn_b, rW1, rb1, rW2, rb2):
    ew = _edge_mlp(edge_attr, eW1, eb1, eW2, eb2)

    pad = EPAD - E
    row_p = jnp.concatenate(
        [edge_index[0], jnp.zeros((pad,), jnp.int32)]).reshape(
            NW, CHUNKS_PER_W, CHUNK)
    col_p = jnp.concatenate(
        [edge_index[1], jnp.zeros((pad,), jnp.int32)]).reshape(
            NW, CHUNKS_PER_W, CHUNK)
    ew_p = jnp.concatenate(
        [ew, jnp.zeros((pad,), jnp.float32)]).reshape(
            NW, CHUNKS_PER_W, CHUNK)

    zdeg = jnp.zeros((NPAD,), jnp.float32)
    znode = jnp.zeros((NPAD, H), jnp.float32)

    deg2 = _get_sc_deg()(col_p, ew_p, zdeg)
    d3 = deg2.reshape(2, NPAD, 1)

    dinv, tp = _embed(x, node_W, node_b, conv_W[0], d3[0], d3[1])

    for i in range(3):
        S = _get_sc_msg()(tp, row_p, col_p, ew_p, znode)
        if i < 2:
            tp = _layer(S, tp, dinv, conv_b[i], bn_g[i], bn_b[i],
                        conv_W[i + 1])
        else:
            out = _final(S, tp, dinv, conv_b[i], bn_g[i], bn_b[i],
                         batch.reshape(1, N), rW1, rb1, rW2, rb2)
    return out
